# baseline (speedup 1.0000x reference)
import functools

import jax
import jax.numpy as jnp
from jax import lax
from jax.experimental import pallas as pl
from jax.experimental.pallas import tpu as pltpu
from jax.experimental.pallas import tpu_sc as plsc

N = 10000
E = 320000
D = 128
DE = 16
G = 64
H = 128
MLP_DIM = 256
C = 10
EPS = 1e-5

NC = 2
NS = 16
NW = NC * NS
BS = 128
KB = 80
CHK = 8
KBO = KB // CHK
EPW = KB * BS
EP = NW * EPW
NP = 10240
RPS = NP // NS
NBUF = 2
ZB = BS

_HIGH = jax.lax.Precision.HIGHEST


def _dot(a, b):
    return jnp.dot(a, b, precision=_HIGH, preferred_element_type=jnp.float32)


def _zero_acc(zrow, rows, acc_sh, s):
    pltpu.sync_copy(zrow, rows)
    for t in range(RPS // ZB):
        base = s * RPS + t * ZB
        pltpu.sync_copy(rows, acc_sh.at[pl.ds(base, ZB)])


def _writeback(rows, acc_sh, out, c, s):
    for t in range(RPS // ZB):
        base = s * RPS + t * ZB
        pltpu.sync_copy(acc_sh.at[pl.ds(base, ZB)], rows)
        pltpu.sync_copy(rows, out.at[c, pl.ds(base, ZB)])


def _pipelined_blocks(load_block, dst_v, acc_sh, bufs, gsems, ssems):
    nbuf = len(bufs)
    gd = [None] * CHK
    sd = [None] * CHK
    for ji in range(min(nbuf, CHK)):
        gd[ji] = load_block(ji, bufs[ji], gsems[ji])
    for ji in range(CHK):
        b = ji % nbuf
        gd[ji].wait()
        sd[ji] = pltpu.async_copy(bufs[b], acc_sh.at[dst_v.at[ji]],
                                  ssems[b], add=True)
        nxt = ji + nbuf
        if nxt < CHK:
            sd[ji].wait()
            gd[nxt] = load_block(nxt, bufs[b], gsems[b])
    for ji in range(CHK - nbuf, CHK):
        sd[ji].wait()


def _sc_gather_body(table, src3, dst3, zrow,
                    out_a,
                    acc_sh, src_v, dst_v, bufs, gsems, ssems):
    c = lax.axis_index("c")
    s = lax.axis_index("s")
    wid = s * NC + c

    _zero_acc(zrow, bufs[0], acc_sh, s)
    plsc.subcore_barrier()

    @pl.loop(0, KBO)
    def _chunk(ko):
        pltpu.sync_copy(src3.at[wid, pl.ds(ko * CHK, CHK)], src_v)
        pltpu.sync_copy(dst3.at[wid, pl.ds(ko * CHK, CHK)], dst_v)

        def load(ji, buf, sem):
            return pltpu.async_copy(table.at[src_v.at[ji]], buf, sem)

        _pipelined_blocks(load, dst_v, acc_sh, bufs, gsems, ssems)

    plsc.subcore_barrier()
    _writeback(bufs[0], acc_sh, out_a, c, s)


def _sc_aug_body(eaug, dst3, zrow,
                 out_u,
                 acc_sh, dst_v, bufs, gsems, ssems):
    c = lax.axis_index("c")
    s = lax.axis_index("s")
    wid = s * NC + c

    _zero_acc(zrow, bufs[0], acc_sh, s)
    plsc.subcore_barrier()

    @pl.loop(0, KBO)
    def _chunk(ko):
        pltpu.sync_copy(dst3.at[wid, pl.ds(ko * CHK, CHK)], dst_v)

        def load(ji, buf, sem):
            base_e = (wid * KB + ko * CHK + ji) * BS
            return pltpu.async_copy(eaug.at[pl.ds(base_e, BS)], buf, sem)

        _pipelined_blocks(load, dst_v, acc_sh, bufs, gsems, ssems)

    plsc.subcore_barrier()
    _writeback(bufs[0], acc_sh, out_u, c, s)


@functools.cache
def _sc_passes():
    mesh = plsc.VectorSubcoreMesh(core_axis_name="c", subcore_axis_name="s",
                                  num_cores=NC, num_subcores=NS)
    rowbufs = tuple(pltpu.VMEM((BS, D), jnp.float32) for _ in range(NBUF))
    sems = tuple(pltpu.SemaphoreType.DMA for _ in range(NBUF))
    gather = pl.kernel(
        _sc_gather_body,
        out_type=jax.ShapeDtypeStruct((NC, NP, D), jnp.float32),
        mesh=mesh,
        scratch_types=(
            pltpu.VMEM_SHARED((NP, D), jnp.float32),
            pltpu.VMEM((CHK, BS), jnp.int32),
            pltpu.VMEM((CHK, BS), jnp.int32),
            rowbufs,
            sems,
            sems,
        ),
    )
    aug = pl.kernel(
        _sc_aug_body,
        out_type=jax.ShapeDtypeStruct((NC, NP, D), jnp.float32),
        mesh=mesh,
        scratch_types=(
            pltpu.VMEM_SHARED((NP, D), jnp.float32),
            pltpu.VMEM((CHK, BS), jnp.int32),
            rowbufs,
            sems,
            sems,
        ),
    )
    return gather, aug


BR = 1024
NBLK = NP // BR
EBR = 4096
ENBLK = EP // EBR


def _rowmask(i):
    rows = i * BR + lax.broadcasted_iota(jnp.int32, (BR, 1), 0)
    return (rows < N).astype(jnp.float32)


def _tc_mp_body(x_ref, a_ref, u_ref, wd_ref, ws_ref, we_ref, b_ref,
                h_ref, stats_ref):
    i = pl.program_id(0)
    x = x_ref[...]
    u = u_ref[0] + u_ref[1]
    deg = u[:, 16:17] + 1.0
    aea = u[:, :DE] + 1.0
    a_sum = a_ref[0] + a_ref[1] + x
    pre = (deg * _dot(x, wd_ref[...]) + _dot(a_sum, ws_ref[...])
           + _dot(aea, we_ref[...]) + deg * b_ref[...])
    h = jnp.maximum(pre, 0.0) * _rowmask(i)
    h_ref[...] = h
    blk = jnp.concatenate([jnp.sum(h, axis=0, keepdims=True),
                           jnp.sum(h * h, axis=0, keepdims=True)], axis=0)

    @pl.when(i == 0)
    def _():
        stats_ref[...] = jnp.zeros_like(stats_ref)

    stats_ref[...] += blk


def _bn_apply(h, stats, g, be, i):
    mu = stats[0:1] * (1.0 / N)
    var = stats[1:2] * (1.0 / N) - mu * mu
    hn = (h - mu) / jnp.sqrt(var + EPS) * g + be
    return jnp.maximum(hn, 0.0) * _rowmask(i)


def _tc_bn1_body(h_ref, stats_ref, g_ref, be_ref, out_ref):
    out_ref[...] = _bn_apply(h_ref[...], stats_ref[...], g_ref[...],
                             be_ref[...], pl.program_id(0))


def _onehot(batch_blk):
    gidx = lax.broadcasted_iota(jnp.int32, (BR, G), 1)
    return (batch_blk == gidx).astype(jnp.float32)


def _tc_bn2_body(h_ref, stats_ref, g_ref, be_ref, batch_ref,
                 out_ref, pool_ref):
    i = pl.program_id(0)
    h2 = _bn_apply(h_ref[...], stats_ref[...], g_ref[...], be_ref[...], i)
    out_ref[...] = h2
    pool_blk = lax.dot_general(_onehot(batch_ref[...]), h2,
                               (((0,), (0,)), ((), ())),
                               preferred_element_type=jnp.float32)

    @pl.when(i == 0)
    def _():
        pool_ref[...] = jnp.zeros_like(pool_ref)

    pool_ref[...] += pool_blk


def _tc_eaug_body(ea_ref, out_ref):
    cols = lax.broadcasted_iota(jnp.int32, (EBR, D), 1)
    ea_wide = jnp.pad(ea_ref[...], ((0, 0), (0, D - DE)))
    out_ref[...] = jnp.where(cols < DE, ea_wide,
                             (cols == DE).astype(jnp.float32))


def _dotd(a, b):
    return jnp.dot(a, b, preferred_element_type=jnp.float32)


def _tc_mlp_body(h2_ref, pool_ref, batch_ref, wf1h_ref, wf1r_ref, bf1_ref,
                 wf2_ref, bf2_ref, out_ref):
    rep = _dotd(_onehot(batch_ref[...]), pool_ref[...])
    no = jnp.maximum(_dotd(h2_ref[...], wf1h_ref[...]) +
                     _dotd(rep, wf1r_ref[...]) + bf1_ref[...], 0.0)
    out_ref[...] = _dotd(no, wf2_ref[...]) + bf2_ref[...]


def _blk(shape, imap):
    return pl.BlockSpec(shape, imap)

_row = lambda i: (i, 0)
_fix = lambda i: (0, 0)
_row3 = lambda i: (0, i, 0)


def _build_tc(interpret=False):
    tc_mp = pl.pallas_call(
        _tc_mp_body,
        grid=(NBLK,),
        in_specs=[
            _blk((BR, D), _row),
            _blk((NC, BR, D), _row3),
            _blk((NC, BR, D), _row3),
            _blk((D, H), _fix),
            _blk((D, H), _fix),
            _blk((DE, H), _fix),
            _blk((1, H), _fix),
        ],
        out_specs=(_blk((BR, H), _row), _blk((2, H), _fix)),
        out_shape=(jax.ShapeDtypeStruct((NP, H), jnp.float32),
                   jax.ShapeDtypeStruct((2, H), jnp.float32)),
        interpret=interpret)

    tc_bn1 = pl.pallas_call(
        _tc_bn1_body,
        grid=(NBLK,),
        in_specs=[
            _blk((BR, H), _row),
            _blk((2, H), _fix),
            _blk((1, H), _fix),
            _blk((1, H), _fix),
        ],
        out_specs=_blk((BR, H), _row),
        out_shape=jax.ShapeDtypeStruct((NP, H), jnp.float32),
        interpret=interpret)

    tc_bn2 = pl.pallas_call(
        _tc_bn2_body,
        grid=(NBLK,),
        in_specs=[
            _blk((BR, H), _row),
            _blk((2, H), _fix),
            _blk((1, H), _fix),
            _blk((1, H), _fix),
            _blk((BR, 1), _row),
        ],
        out_specs=(_blk((BR, H), _row), _blk((G, H), _fix)),
        out_shape=(jax.ShapeDtypeStruct((NP, H), jnp.float32),
                   jax.ShapeDtypeStruct((G, H), jnp.float32)),
        interpret=interpret)

    tc_eaug = pl.pallas_call(
        _tc_eaug_body,
        grid=(ENBLK,),
        in_specs=[_blk((EBR, DE), _row)],
        out_specs=_blk((EBR, D), _row),
        out_shape=jax.ShapeDtypeStruct((EP, D), jnp.float32),
        interpret=interpret)

    tc_mlp = pl.pallas_call(
        _tc_mlp_body,
        grid=(NBLK,),
        in_specs=[
            _blk((BR, H), _row),
            _blk((G, H), _fix),
            _blk((BR, 1), _row),
            _blk((H, MLP_DIM), _fix),
            _blk((H, MLP_DIM), _fix),
            _blk((1, MLP_DIM), _fix),
            _blk((MLP_DIM, C), _fix),
            _blk((1, C), _fix),
        ],
        out_specs=_blk((BR, C), _row),
        out_shape=jax.ShapeDtypeStruct((NP, C), jnp.float32),
        interpret=interpret)
    return tc_mp, tc_bn1, tc_bn2, tc_mlp, tc_eaug


_tc_mp, _tc_bn1, _tc_bn2, _tc_mlp, _tc_eaug = _build_tc()


def kernel(x, edge_index, edge_attr, batch, mask,
           W1, b1, g1, be1, W2, b2, g2, be2, Wf1, bf1, Wf2, bf2):
    del mask
    src = edge_index[0]
    dst = edge_index[1]
    pad_e = EP - E
    pad_ids = jnp.arange(pad_e, dtype=jnp.int32)
    src3 = jnp.concatenate([src, pad_ids % N]).reshape(NW, KB, BS)
    dst3 = jnp.concatenate(
        [dst, N + pad_ids % (NP - N)]).reshape(NW, KB, BS)
    eaug = _tc_eaug(jnp.pad(edge_attr, ((0, pad_e), (0, 0))))

    x_pad = jnp.concatenate([x, jnp.zeros((NP - N, D), jnp.float32)])
    batch2 = jnp.concatenate(
        [batch, jnp.full((NP - N,), G, jnp.int32)]).reshape(NP, 1)

    zrow = jnp.zeros((BS, D), jnp.float32)

    wd1 = W1[:, :D].T
    ws1 = W1[:, D:2 * D].T
    we1 = W1[:, 2 * D:].T
    wd2 = W2[:, :H].T
    ws2 = W2[:, H:2 * H].T
    we2 = W2[:, 2 * H:].T
    wf1h = Wf1[:, :H].T
    wf1r = Wf1[:, H:].T
    wf2 = Wf2.T

    sc_gather, sc_aug = _sc_passes()
    u1 = sc_aug(eaug, dst3, zrow)
    a1 = sc_gather(x_pad, src3, dst3, zrow)
    hp1, stats1 = _tc_mp(x_pad, a1, u1, wd1, ws1, we1, b1.reshape(1, H))
    h1 = _tc_bn1(hp1, stats1, g1.reshape(1, H), be1.reshape(1, H))
    a2 = sc_gather(h1, src3, dst3, zrow)
    hp2, stats2 = _tc_mp(h1, a2, u1, wd2, ws2, we2, b2.reshape(1, H))
    h2, pool = _tc_bn2(hp2, stats2, g2.reshape(1, H), be2.reshape(1, H),
                       batch2)
    out = _tc_mlp(h2, pool, batch2, wf1h, wf1r, bf1.reshape(1, MLP_DIM),
                  wf2, bf2.reshape(1, C))
    return out[:N]


# speedup vs baseline: 1.1351x; 1.1351x over previous
import functools

import jax
import jax.numpy as jnp
from jax import lax
from jax.experimental import pallas as pl
from jax.experimental.pallas import tpu as pltpu
from jax.experimental.pallas import tpu_sc as plsc

N = 10000
E = 320000
D = 128
DE = 16
G = 64
H = 128
MLP_DIM = 256
C = 10
EPS = 1e-5

NC = 2
NS = 16
NW = NC * NS
BS = 128
KB = 80
CHK = 8
KBO = KB // CHK
EPW = KB * BS
EP = NW * EPW
NP = 10240
RPS = NP // NS
NBUF = 2
ZB = BS

_HIGH = jax.lax.Precision.HIGHEST


def _dot(a, b):
    return jnp.dot(a, b, precision=_HIGH, preferred_element_type=jnp.float32)


def _zero_acc(zrow, rows, acc_sh, s):
    pltpu.sync_copy(zrow, rows)
    for t in range(RPS // ZB):
        base = s * RPS + t * ZB
        pltpu.sync_copy(rows, acc_sh.at[pl.ds(base, ZB)])


def _writeback(rows, acc_sh, out, c, s):
    for t in range(RPS // ZB):
        base = s * RPS + t * ZB
        pltpu.sync_copy(acc_sh.at[pl.ds(base, ZB)], rows)
        pltpu.sync_copy(rows, out.at[c, pl.ds(base, ZB)])


def _pipelined_blocks(load_block, dst_v, acc_sh, bufs, gsems, ssems):
    nbuf = len(bufs)
    gd = [None] * CHK
    sd = [None] * CHK
    for ji in range(min(nbuf, CHK)):
        gd[ji] = load_block(ji, bufs[ji], gsems[ji])
    for ji in range(CHK):
        b = ji % nbuf
        gd[ji].wait()
        sd[ji] = pltpu.async_copy(bufs[b], acc_sh.at[dst_v.at[ji]],
                                  ssems[b], add=True)
        nxt = ji + nbuf
        if nxt < CHK:
            sd[ji].wait()
            gd[nxt] = load_block(nxt, bufs[b], gsems[b])
    for ji in range(CHK - nbuf, CHK):
        sd[ji].wait()


def _sc_gather_body(table, src3, dst3, zrow,
                    out_a,
                    acc_sh, src_v, dst_v, bufs, gsems, ssems):
    c = lax.axis_index("c")
    s = lax.axis_index("s")
    wid = s * NC + c

    _zero_acc(zrow, bufs[0], acc_sh, s)
    plsc.subcore_barrier()

    @pl.loop(0, KBO)
    def _chunk(ko):
        pltpu.sync_copy(src3.at[wid, pl.ds(ko * CHK, CHK)], src_v)
        pltpu.sync_copy(dst3.at[wid, pl.ds(ko * CHK, CHK)], dst_v)

        def load(ji, buf, sem):
            return pltpu.async_copy(table.at[src_v.at[ji]], buf, sem)

        _pipelined_blocks(load, dst_v, acc_sh, bufs, gsems, ssems)

    plsc.subcore_barrier()
    _writeback(bufs[0], acc_sh, out_a, c, s)


def _sc_aug_body(eaug, dst3, zrow,
                 out_u,
                 acc_sh, dst_v, bufs, gsems, ssems):
    c = lax.axis_index("c")
    s = lax.axis_index("s")
    wid = s * NC + c

    _zero_acc(zrow, bufs[0], acc_sh, s)
    plsc.subcore_barrier()

    @pl.loop(0, KBO)
    def _chunk(ko):
        pltpu.sync_copy(dst3.at[wid, pl.ds(ko * CHK, CHK)], dst_v)

        def load(ji, buf, sem):
            base_e = (wid * KB + ko * CHK + ji) * BS
            return pltpu.async_copy(eaug.at[pl.ds(base_e, BS)], buf, sem)

        _pipelined_blocks(load, dst_v, acc_sh, bufs, gsems, ssems)

    plsc.subcore_barrier()
    _writeback(bufs[0], acc_sh, out_u, c, s)


@functools.cache
def _sc_passes():
    mesh = plsc.VectorSubcoreMesh(core_axis_name="c", subcore_axis_name="s",
                                  num_cores=NC, num_subcores=NS)
    rowbufs = tuple(pltpu.VMEM((BS, D), jnp.float32) for _ in range(NBUF))
    sems = tuple(pltpu.SemaphoreType.DMA for _ in range(NBUF))
    gather = pl.kernel(
        _sc_gather_body,
        out_type=jax.ShapeDtypeStruct((NC, NP, D), jnp.float32),
        mesh=mesh,
        scratch_types=(
            pltpu.VMEM_SHARED((NP, D), jnp.float32),
            pltpu.VMEM((CHK, BS), jnp.int32),
            pltpu.VMEM((CHK, BS), jnp.int32),
            rowbufs,
            sems,
            sems,
        ),
    )
    aug = pl.kernel(
        _sc_aug_body,
        out_type=jax.ShapeDtypeStruct((NC, NP, D), jnp.float32),
        mesh=mesh,
        scratch_types=(
            pltpu.VMEM_SHARED((NP, D), jnp.float32),
            pltpu.VMEM((CHK, BS), jnp.int32),
            rowbufs,
            sems,
            sems,
        ),
    )
    return gather, aug


BR = 1024
NBLK = NP // BR
EBR = 4096
ENBLK = EP // EBR


def _rowmask(i):
    rows = i * BR + lax.broadcasted_iota(jnp.int32, (BR, 1), 0)
    return (rows < N).astype(jnp.float32)


def _tc_mp_body(x_ref, a_ref, u_ref, wd_ref, ws_ref, we_ref, b_ref,
                h_ref, stats_ref):
    i = pl.program_id(0)
    x = x_ref[...]
    u = u_ref[0] + u_ref[1]
    deg = u[:, 16:17] + 1.0
    aea = u[:, :DE] + 1.0
    a_sum = a_ref[0] + a_ref[1] + x
    pre = (deg * _dot(x, wd_ref[...]) + _dot(a_sum, ws_ref[...])
           + _dot(aea, we_ref[...]) + deg * b_ref[...])
    h = jnp.maximum(pre, 0.0) * _rowmask(i)
    h_ref[...] = h
    blk = jnp.concatenate([jnp.sum(h, axis=0, keepdims=True),
                           jnp.sum(h * h, axis=0, keepdims=True)], axis=0)

    @pl.when(i == 0)
    def _():
        stats_ref[...] = jnp.zeros_like(stats_ref)

    stats_ref[...] += blk


def _bn_apply(h, stats, g, be, i):
    mu = stats[0:1] * (1.0 / N)
    var = stats[1:2] * (1.0 / N) - mu * mu
    hn = (h - mu) / jnp.sqrt(var + EPS) * g + be
    return jnp.maximum(hn, 0.0) * _rowmask(i)


def _tc_bn1_body(h_ref, stats_ref, g_ref, be_ref, out_ref):
    out_ref[...] = _bn_apply(h_ref[...], stats_ref[...], g_ref[...],
                             be_ref[...], pl.program_id(0))


def _onehot(batch_blk):
    gidx = lax.broadcasted_iota(jnp.int32, (BR, G), 1)
    return (batch_blk == gidx).astype(jnp.float32)


def _tc_bn2_body(h_ref, stats_ref, g_ref, be_ref, batch_ref,
                 out_ref, pool_ref):
    i = pl.program_id(0)
    h2 = _bn_apply(h_ref[...], stats_ref[...], g_ref[...], be_ref[...], i)
    out_ref[...] = h2
    pool_blk = lax.dot_general(_onehot(batch_ref[...]), h2,
                               (((0,), (0,)), ((), ())),
                               preferred_element_type=jnp.float32)

    @pl.when(i == 0)
    def _():
        pool_ref[...] = jnp.zeros_like(pool_ref)

    pool_ref[...] += pool_blk


def _tc_eaug_body(ea_ref, out_ref):
    cols = lax.broadcasted_iota(jnp.int32, (EBR, D), 1)
    ea_wide = jnp.pad(ea_ref[...], ((0, 0), (0, D - DE)))
    out_ref[...] = jnp.where(cols < DE, ea_wide,
                             (cols == DE).astype(jnp.float32))


def _dotd(a, b):
    return jnp.dot(a, b, preferred_element_type=jnp.float32)


def _tc_mlp_body(h2_ref, pool_ref, batch_ref, wf1h_ref, wf1r_ref, bf1_ref,
                 wf2_ref, bf2_ref, out_ref):
    rep = _dotd(_onehot(batch_ref[...]), pool_ref[...])
    no = jnp.maximum(_dotd(h2_ref[...], wf1h_ref[...]) +
                     _dotd(rep, wf1r_ref[...]) + bf1_ref[...], 0.0)
    out_ref[...] = _dotd(no, wf2_ref[...]) + bf2_ref[...]


def _blk(shape, imap):
    return pl.BlockSpec(shape, imap)

_row = lambda i: (i, 0)
_fix = lambda i: (0, 0)
_row3 = lambda i: (0, i, 0)


def _build_tc(interpret=False):
    tc_mp = pl.pallas_call(
        _tc_mp_body,
        grid=(NBLK,),
        in_specs=[
            _blk((BR, D), _row),
            _blk((NC, BR, D), _row3),
            _blk((NC, BR, D), _row3),
            _blk((D, H), _fix),
            _blk((D, H), _fix),
            _blk((DE, H), _fix),
            _blk((1, H), _fix),
        ],
        out_specs=(_blk((BR, H), _row), _blk((2, H), _fix)),
        out_shape=(jax.ShapeDtypeStruct((NP, H), jnp.float32),
                   jax.ShapeDtypeStruct((2, H), jnp.float32)),
        interpret=interpret)

    tc_bn1 = pl.pallas_call(
        _tc_bn1_body,
        grid=(NBLK,),
        in_specs=[
            _blk((BR, H), _row),
            _blk((2, H), _fix),
            _blk((1, H), _fix),
            _blk((1, H), _fix),
        ],
        out_specs=_blk((BR, H), _row),
        out_shape=jax.ShapeDtypeStruct((NP, H), jnp.float32),
        interpret=interpret)

    tc_bn2 = pl.pallas_call(
        _tc_bn2_body,
        grid=(NBLK,),
        in_specs=[
            _blk((BR, H), _row),
            _blk((2, H), _fix),
            _blk((1, H), _fix),
            _blk((1, H), _fix),
            _blk((BR, 1), _row),
        ],
        out_specs=(_blk((BR, H), _row), _blk((G, H), _fix)),
        out_shape=(jax.ShapeDtypeStruct((NP, H), jnp.float32),
                   jax.ShapeDtypeStruct((G, H), jnp.float32)),
        interpret=interpret)

    tc_eaug = pl.pallas_call(
        _tc_eaug_body,
        grid=(ENBLK,),
        in_specs=[_blk((EBR, DE), _row)],
        out_specs=_blk((EBR, D), _row),
        out_shape=jax.ShapeDtypeStruct((EP, D), jnp.float32),
        interpret=interpret)

    tc_mlp = pl.pallas_call(
        _tc_mlp_body,
        grid=(NBLK,),
        in_specs=[
            _blk((BR, H), _row),
            _blk((G, H), _fix),
            _blk((BR, 1), _row),
            _blk((H, MLP_DIM), _fix),
            _blk((H, MLP_DIM), _fix),
            _blk((1, MLP_DIM), _fix),
            _blk((MLP_DIM, C), _fix),
            _blk((1, C), _fix),
        ],
        out_specs=_blk((BR, C), _row),
        out_shape=jax.ShapeDtypeStruct((NP, C), jnp.float32),
        interpret=interpret)
    return tc_mp, tc_bn1, tc_bn2, tc_mlp, tc_eaug


_tc_mp, _tc_bn1, _tc_bn2, _tc_mlp, _tc_eaug = _build_tc()


def kernel(x, edge_index, edge_attr, batch, mask,
           W1, b1, g1, be1, W2, b2, g2, be2, Wf1, bf1, Wf2, bf2):
    del mask
    src = edge_index[0]
    dst = edge_index[1]
    pad_e = EP - E
    pad_ids = jnp.arange(pad_e, dtype=jnp.int32)
    src3 = jnp.concatenate([src, pad_ids % N]).reshape(NW, KB, BS)
    dst3 = jnp.concatenate(
        [dst, N + pad_ids % (NP - N)]).reshape(NW, KB, BS)
    eaug = jnp.pad(jnp.concatenate(
        [edge_attr, jnp.ones((E, 1), jnp.float32)], axis=1),
        ((0, pad_e), (0, D - DE - 1)))

    x_pad = jnp.concatenate([x, jnp.zeros((NP - N, D), jnp.float32)])
    batch2 = jnp.concatenate(
        [batch, jnp.full((NP - N,), G, jnp.int32)]).reshape(NP, 1)

    zrow = jnp.zeros((BS, D), jnp.float32)

    wd1 = W1[:, :D].T
    ws1 = W1[:, D:2 * D].T
    we1 = W1[:, 2 * D:].T
    wd2 = W2[:, :H].T
    ws2 = W2[:, H:2 * H].T
    we2 = W2[:, 2 * H:].T
    wf1h = Wf1[:, :H].T
    wf1r = Wf1[:, H:].T
    wf2 = Wf2.T

    sc_gather, sc_aug = _sc_passes()
    u1 = sc_aug(eaug, dst3, zrow)
    a1 = sc_gather(x_pad, src3, dst3, zrow)
    hp1, stats1 = _tc_mp(x_pad, a1, u1, wd1, ws1, we1, b1.reshape(1, H))
    h1 = _tc_bn1(hp1, stats1, g1.reshape(1, H), be1.reshape(1, H))
    a2 = sc_gather(h1, src3, dst3, zrow)
    hp2, stats2 = _tc_mp(h1, a2, u1, wd2, ws2, we2, b2.reshape(1, H))
    h2, pool = _tc_bn2(hp2, stats2, g2.reshape(1, H), be2.reshape(1, H),
                       batch2)
    out = _tc_mlp(h2, pool, batch2, wf1h, wf1r, bf1.reshape(1, MLP_DIM),
                  wf2, bf2.reshape(1, C))
    return out[:N]


# speedup vs baseline: 1.1575x; 1.0197x over previous
import functools

import jax
import jax.numpy as jnp
from jax import lax
from jax.experimental import pallas as pl
from jax.experimental.pallas import tpu as pltpu
from jax.experimental.pallas import tpu_sc as plsc

N = 10000
E = 320000
D = 128
DE = 16
G = 64
H = 128
MLP_DIM = 256
C = 10
EPS = 1e-5

NC = 2
NS = 16
NW = NC * NS
BS = 128
KB = 80
CHK = 8
KBO = KB // CHK
EPW = KB * BS
EP = NW * EPW
NP = 10240
RPS = NP // NS
NBUF = 2
ZB = BS

_HIGH = jax.lax.Precision.HIGHEST


def _dot(a, b):
    return jnp.dot(a, b, precision=_HIGH, preferred_element_type=jnp.float32)


def _dotd(a, b):
    return jnp.dot(a, b, preferred_element_type=jnp.float32)


def _zero_acc(zrow, rows, acc_sh, s):
    pltpu.sync_copy(zrow, rows)
    for t in range(RPS // ZB):
        base = s * RPS + t * ZB
        pltpu.sync_copy(rows, acc_sh.at[pl.ds(base, ZB)])


def _writeback(rows, acc_sh, out, c, s):
    for t in range(RPS // ZB):
        base = s * RPS + t * ZB
        pltpu.sync_copy(acc_sh.at[pl.ds(base, ZB)], rows)
        pltpu.sync_copy(rows, out.at[c, pl.ds(base, ZB)])


def _pipelined_blocks(load_block, dst_v, acc_sh, bufs, gsems, ssems):
    nbuf = len(bufs)
    gd = [None] * CHK
    sd = [None] * CHK
    for ji in range(min(nbuf, CHK)):
        gd[ji] = load_block(ji, bufs[ji], gsems[ji])
    for ji in range(CHK):
        b = ji % nbuf
        gd[ji].wait()
        sd[ji] = pltpu.async_copy(bufs[b], acc_sh.at[dst_v.at[ji]],
                                  ssems[b], add=True)
        nxt = ji + nbuf
        if nxt < CHK:
            sd[ji].wait()
            gd[nxt] = load_block(nxt, bufs[b], gsems[b])
    for ji in range(CHK - nbuf, CHK):
        sd[ji].wait()


def _sc_gather_body(table, src3, dst3, zrow,
                    out_a,
                    acc_sh, src_v, dst_v, bufs, gsems, ssems):
    c = lax.axis_index("c")
    s = lax.axis_index("s")
    wid = s * NC + c

    _zero_acc(zrow, bufs[0], acc_sh, s)
    plsc.subcore_barrier()

    @pl.loop(0, KBO)
    def _chunk(ko):
        pltpu.sync_copy(src3.at[wid, pl.ds(ko * CHK, CHK)], src_v)
        pltpu.sync_copy(dst3.at[wid, pl.ds(ko * CHK, CHK)], dst_v)

        def load(ji, buf, sem):
            return pltpu.async_copy(table.at[src_v.at[ji]], buf, sem)

        _pipelined_blocks(load, dst_v, acc_sh, bufs, gsems, ssems)

    plsc.subcore_barrier()
    _writeback(bufs[0], acc_sh, out_a, c, s)


def _sc_aug_body(eaug, dst3, zrow,
                 out_u,
                 acc_sh, dst_v, bufs, gsems, ssems):
    c = lax.axis_index("c")
    s = lax.axis_index("s")
    wid = s * NC + c

    _zero_acc(zrow, bufs[0], acc_sh, s)
    plsc.subcore_barrier()

    @pl.loop(0, KBO)
    def _chunk(ko):
        pltpu.sync_copy(dst3.at[wid, pl.ds(ko * CHK, CHK)], dst_v)

        def load(ji, buf, sem):
            base_e = (wid * KB + ko * CHK + ji) * BS
            return pltpu.async_copy(eaug.at[pl.ds(base_e, BS)], buf, sem)

        _pipelined_blocks(load, dst_v, acc_sh, bufs, gsems, ssems)

    plsc.subcore_barrier()
    _writeback(bufs[0], acc_sh, out_u, c, s)


@functools.cache
def _sc_passes():
    mesh = plsc.VectorSubcoreMesh(core_axis_name="c", subcore_axis_name="s",
                                  num_cores=NC, num_subcores=NS)
    rowbufs = tuple(pltpu.VMEM((BS, D), jnp.float32) for _ in range(NBUF))
    sems = tuple(pltpu.SemaphoreType.DMA for _ in range(NBUF))
    gather = pl.kernel(
        _sc_gather_body,
        out_type=jax.ShapeDtypeStruct((NC, NP, D), jnp.float32),
        mesh=mesh,
        scratch_types=(
            pltpu.VMEM_SHARED((NP, D), jnp.float32),
            pltpu.VMEM((CHK, BS), jnp.int32),
            pltpu.VMEM((CHK, BS), jnp.int32),
            rowbufs,
            sems,
            sems,
        ),
    )
    aug = pl.kernel(
        _sc_aug_body,
        out_type=jax.ShapeDtypeStruct((NC, NP, D), jnp.float32),
        mesh=mesh,
        scratch_types=(
            pltpu.VMEM_SHARED((NP, D), jnp.float32),
            pltpu.VMEM((CHK, BS), jnp.int32),
            rowbufs,
            sems,
            sems,
        ),
    )
    return gather, aug


BR = 1024
NBLK = NP // BR
EBR = 4096
ENBLK = EP // EBR


def _rowmask(i):
    rows = i * BR + lax.broadcasted_iota(jnp.int32, (BR, 1), 0)
    return (rows < N).astype(jnp.float32)


def _tc_mp_body(x_ref, a_ref, u_ref, wd_ref, ws_ref, we_ref, b_ref,
                h_ref, stats_ref):
    i = pl.program_id(0)
    x = x_ref[...]
    u = u_ref[0] + u_ref[1]
    deg = u[:, 16:17] + 1.0
    aea = u[:, :DE] + 1.0
    a_sum = a_ref[0] + a_ref[1] + x
    pre = (deg * _dotd(x, wd_ref[...]) + _dotd(a_sum, ws_ref[...])
           + _dotd(aea, we_ref[...]) + deg * b_ref[...])
    h = jnp.maximum(pre, 0.0) * _rowmask(i)
    h_ref[...] = h
    blk = jnp.concatenate([jnp.sum(h, axis=0, keepdims=True),
                           jnp.sum(h * h, axis=0, keepdims=True)], axis=0)

    @pl.when(i == 0)
    def _():
        stats_ref[...] = jnp.zeros_like(stats_ref)

    stats_ref[...] += blk


def _bn_apply(h, stats, g, be, i):
    mu = stats[0:1] * (1.0 / N)
    var = stats[1:2] * (1.0 / N) - mu * mu
    hn = (h - mu) / jnp.sqrt(var + EPS) * g + be
    return jnp.maximum(hn, 0.0) * _rowmask(i)


def _tc_bn1_body(h_ref, stats_ref, g_ref, be_ref, out_ref):
    out_ref[...] = _bn_apply(h_ref[...], stats_ref[...], g_ref[...],
                             be_ref[...], pl.program_id(0))


def _onehot(batch_blk):
    gidx = lax.broadcasted_iota(jnp.int32, (BR, G), 1)
    return (batch_blk == gidx).astype(jnp.float32)


def _tc_bn2_body(h_ref, stats_ref, g_ref, be_ref, batch_ref,
                 out_ref, pool_ref):
    i = pl.program_id(0)
    h2 = _bn_apply(h_ref[...], stats_ref[...], g_ref[...], be_ref[...], i)
    out_ref[...] = h2
    pool_blk = lax.dot_general(_onehot(batch_ref[...]), h2,
                               (((0,), (0,)), ((), ())),
                               preferred_element_type=jnp.float32)

    @pl.when(i == 0)
    def _():
        pool_ref[...] = jnp.zeros_like(pool_ref)

    pool_ref[...] += pool_blk


def _tc_eaug_body(ea_ref, out_ref):
    cols = lax.broadcasted_iota(jnp.int32, (EBR, D), 1)
    ea_wide = jnp.pad(ea_ref[...], ((0, 0), (0, D - DE)))
    out_ref[...] = jnp.where(cols < DE, ea_wide,
                             (cols == DE).astype(jnp.float32))


def _tc_mlp_body(h2_ref, pool_ref, batch_ref, wf1h_ref, wf1r_ref, bf1_ref,
                 wf2_ref, bf2_ref, out_ref):
    rep = _dotd(_onehot(batch_ref[...]), pool_ref[...])
    no = jnp.maximum(_dotd(h2_ref[...], wf1h_ref[...]) +
                     _dotd(rep, wf1r_ref[...]) + bf1_ref[...], 0.0)
    out_ref[...] = _dotd(no, wf2_ref[...]) + bf2_ref[...]


def _blk(shape, imap):
    return pl.BlockSpec(shape, imap)

_row = lambda i: (i, 0)
_fix = lambda i: (0, 0)
_row3 = lambda i: (0, i, 0)


def _build_tc(interpret=False):
    tc_mp = pl.pallas_call(
        _tc_mp_body,
        grid=(NBLK,),
        in_specs=[
            _blk((BR, D), _row),
            _blk((NC, BR, D), _row3),
            _blk((NC, BR, D), _row3),
            _blk((D, H), _fix),
            _blk((D, H), _fix),
            _blk((DE, H), _fix),
            _blk((1, H), _fix),
        ],
        out_specs=(_blk((BR, H), _row), _blk((2, H), _fix)),
        out_shape=(jax.ShapeDtypeStruct((NP, H), jnp.float32),
                   jax.ShapeDtypeStruct((2, H), jnp.float32)),
        interpret=interpret)

    tc_bn1 = pl.pallas_call(
        _tc_bn1_body,
        grid=(NBLK,),
        in_specs=[
            _blk((BR, H), _row),
            _blk((2, H), _fix),
            _blk((1, H), _fix),
            _blk((1, H), _fix),
        ],
        out_specs=_blk((BR, H), _row),
        out_shape=jax.ShapeDtypeStruct((NP, H), jnp.float32),
        interpret=interpret)

    tc_bn2 = pl.pallas_call(
        _tc_bn2_body,
        grid=(NBLK,),
        in_specs=[
            _blk((BR, H), _row),
            _blk((2, H), _fix),
            _blk((1, H), _fix),
            _blk((1, H), _fix),
            _blk((BR, 1), _row),
        ],
        out_specs=(_blk((BR, H), _row), _blk((G, H), _fix)),
        out_shape=(jax.ShapeDtypeStruct((NP, H), jnp.float32),
                   jax.ShapeDtypeStruct((G, H), jnp.float32)),
        interpret=interpret)

    tc_eaug = pl.pallas_call(
        _tc_eaug_body,
        grid=(ENBLK,),
        in_specs=[_blk((EBR, DE), _row)],
        out_specs=_blk((EBR, D), _row),
        out_shape=jax.ShapeDtypeStruct((EP, D), jnp.float32),
        interpret=interpret)

    tc_mlp = pl.pallas_call(
        _tc_mlp_body,
        grid=(NBLK,),
        in_specs=[
            _blk((BR, H), _row),
            _blk((G, H), _fix),
            _blk((BR, 1), _row),
            _blk((H, MLP_DIM), _fix),
            _blk((H, MLP_DIM), _fix),
            _blk((1, MLP_DIM), _fix),
            _blk((MLP_DIM, C), _fix),
            _blk((1, C), _fix),
        ],
        out_specs=_blk((BR, C), _row),
        out_shape=jax.ShapeDtypeStruct((NP, C), jnp.float32),
        interpret=interpret)
    return tc_mp, tc_bn1, tc_bn2, tc_mlp, tc_eaug


_tc_mp, _tc_bn1, _tc_bn2, _tc_mlp, _tc_eaug = _build_tc()


def kernel(x, edge_index, edge_attr, batch, mask,
           W1, b1, g1, be1, W2, b2, g2, be2, Wf1, bf1, Wf2, bf2):
    del mask
    src = edge_index[0]
    dst = edge_index[1]
    pad_e = EP - E
    pad_ids = jnp.arange(pad_e, dtype=jnp.int32)
    src3 = jnp.concatenate([src, pad_ids % N]).reshape(NW, KB, BS)
    dst3 = jnp.concatenate(
        [dst, N + pad_ids % (NP - N)]).reshape(NW, KB, BS)
    eaug = jnp.pad(jnp.concatenate(
        [edge_attr, jnp.ones((E, 1), jnp.float32)], axis=1),
        ((0, pad_e), (0, D - DE - 1)))

    x_pad = jnp.concatenate([x, jnp.zeros((NP - N, D), jnp.float32)])
    batch2 = jnp.concatenate(
        [batch, jnp.full((NP - N,), G, jnp.int32)]).reshape(NP, 1)

    zrow = jnp.zeros((BS, D), jnp.float32)

    wd1 = W1[:, :D].T
    ws1 = W1[:, D:2 * D].T
    we1 = W1[:, 2 * D:].T
    wd2 = W2[:, :H].T
    ws2 = W2[:, H:2 * H].T
    we2 = W2[:, 2 * H:].T
    wf1h = Wf1[:, :H].T
    wf1r = Wf1[:, H:].T
    wf2 = Wf2.T

    sc_gather, sc_aug = _sc_passes()
    u1 = sc_aug(eaug, dst3, zrow)
    a1 = sc_gather(x_pad, src3, dst3, zrow)
    hp1, stats1 = _tc_mp(x_pad, a1, u1, wd1, ws1, we1, b1.reshape(1, H))
    h1 = _tc_bn1(hp1, stats1, g1.reshape(1, H), be1.reshape(1, H))
    a2 = sc_gather(h1, src3, dst3, zrow)
    hp2, stats2 = _tc_mp(h1, a2, u1, wd2, ws2, we2, b2.reshape(1, H))
    h2, pool = _tc_bn2(hp2, stats2, g2.reshape(1, H), be2.reshape(1, H),
                       batch2)
    out = _tc_mlp(h2, pool, batch2, wf1h, wf1r, bf1.reshape(1, MLP_DIM),
                  wf2, bf2.reshape(1, C))
    return out[:N]


# speedup vs baseline: 1.1591x; 1.0015x over previous
import functools

import jax
import jax.numpy as jnp
from jax import lax
from jax.experimental import pallas as pl
from jax.experimental.pallas import tpu as pltpu
from jax.experimental.pallas import tpu_sc as plsc

N = 10000
E = 320000
D = 128
DE = 16
G = 64
H = 128
MLP_DIM = 256
C = 10
EPS = 1e-5

NC = 2
NS = 16
NW = NC * NS
BS = 128
KB = 80
CHK = 8
KBO = KB // CHK
EPW = KB * BS
EP = NW * EPW
NP = 10240
RPS = NP // NS
NBUF = 2
ZB = BS

_HIGH = jax.lax.Precision.HIGHEST


def _dot(a, b):
    return jnp.dot(a, b, precision=_HIGH, preferred_element_type=jnp.float32)


def _dotd(a, b):
    return jnp.dot(a, b, preferred_element_type=jnp.float32)


def _zero_acc(zrow, rows, acc_sh, s):
    pltpu.sync_copy(zrow, rows)
    for t in range(RPS // ZB):
        base = s * RPS + t * ZB
        pltpu.sync_copy(rows, acc_sh.at[pl.ds(base, ZB)])


def _writeback(rows, acc_sh, out, c, s):
    for t in range(RPS // ZB):
        base = s * RPS + t * ZB
        pltpu.sync_copy(acc_sh.at[pl.ds(base, ZB)], rows)
        pltpu.sync_copy(rows, out.at[c, pl.ds(base, ZB)])


def _pipelined_blocks(load_block, dst_v, acc_sh, bufs, gsems, ssems):
    nbuf = len(bufs)
    gd = [None] * CHK
    sd = [None] * CHK
    for ji in range(min(nbuf, CHK)):
        gd[ji] = load_block(ji, bufs[ji], gsems[ji])
    for ji in range(CHK):
        b = ji % nbuf
        gd[ji].wait()
        sd[ji] = pltpu.async_copy(bufs[b], acc_sh.at[dst_v.at[ji]],
                                  ssems[b], add=True)
        nxt = ji + nbuf
        if nxt < CHK:
            sd[ji].wait()
            gd[nxt] = load_block(nxt, bufs[b], gsems[b])
    for ji in range(CHK - nbuf, CHK):
        sd[ji].wait()


def _sc_gather_body(table, src3, dst3, zrow,
                    out_a,
                    acc_sh, src_v, dst_v, bufs, gsems, ssems):
    c = lax.axis_index("c")
    s = lax.axis_index("s")
    wid = s * NC + c

    _zero_acc(zrow, bufs[0], acc_sh, s)
    plsc.subcore_barrier()

    @pl.loop(0, KBO)
    def _chunk(ko):
        pltpu.sync_copy(src3.at[wid, pl.ds(ko * CHK, CHK)], src_v)
        pltpu.sync_copy(dst3.at[wid, pl.ds(ko * CHK, CHK)], dst_v)

        def load(ji, buf, sem):
            return pltpu.async_copy(table.at[src_v.at[ji]], buf, sem)

        _pipelined_blocks(load, dst_v, acc_sh, bufs, gsems, ssems)

    plsc.subcore_barrier()
    _writeback(bufs[0], acc_sh, out_a, c, s)


def _sc_aug_body(eaug, dst3, zrow,
                 out_u,
                 acc_sh, dst_v, bufs, gsems, ssems):
    c = lax.axis_index("c")
    s = lax.axis_index("s")
    wid = s * NC + c

    _zero_acc(zrow, bufs[0], acc_sh, s)
    plsc.subcore_barrier()

    @pl.loop(0, KBO)
    def _chunk(ko):
        pltpu.sync_copy(dst3.at[wid, pl.ds(ko * CHK, CHK)], dst_v)

        def load(ji, buf, sem):
            base_e = (wid * KB + ko * CHK + ji) * BS
            return pltpu.async_copy(eaug.at[pl.ds(base_e, BS)], buf, sem)

        _pipelined_blocks(load, dst_v, acc_sh, bufs, gsems, ssems)

    plsc.subcore_barrier()
    _writeback(bufs[0], acc_sh, out_u, c, s)


@functools.cache
def _sc_passes():
    mesh = plsc.VectorSubcoreMesh(core_axis_name="c", subcore_axis_name="s",
                                  num_cores=NC, num_subcores=NS)
    rowbufs = tuple(pltpu.VMEM((BS, D), jnp.float32) for _ in range(NBUF))
    sems = tuple(pltpu.SemaphoreType.DMA for _ in range(NBUF))
    gather = pl.kernel(
        _sc_gather_body,
        out_type=jax.ShapeDtypeStruct((NC, NP, D), jnp.float32),
        mesh=mesh,
        scratch_types=(
            pltpu.VMEM_SHARED((NP, D), jnp.float32),
            pltpu.VMEM((CHK, BS), jnp.int32),
            pltpu.VMEM((CHK, BS), jnp.int32),
            rowbufs,
            sems,
            sems,
        ),
    )
    aug = pl.kernel(
        _sc_aug_body,
        out_type=jax.ShapeDtypeStruct((NC, NP, D), jnp.float32),
        mesh=mesh,
        scratch_types=(
            pltpu.VMEM_SHARED((NP, D), jnp.float32),
            pltpu.VMEM((CHK, BS), jnp.int32),
            rowbufs,
            sems,
            sems,
        ),
    )
    return gather, aug


BR = 1024
NBLK = NP // BR


def _rowmask(i):
    rows = i * BR + lax.broadcasted_iota(jnp.int32, (BR, 1), 0)
    return (rows < N).astype(jnp.float32)


def _tc_mp_body(x_ref, a_ref, u_ref, wd_ref, ws_ref, we_ref, b_ref,
                h_ref, stats_ref):
    i = pl.program_id(0)
    x = x_ref[...]
    u = u_ref[0] + u_ref[1]
    deg = u[:, 16:17] + 1.0
    aea = u[:, :DE] + 1.0
    a_sum = a_ref[0] + a_ref[1] + x
    pre = (deg * _dotd(x, wd_ref[...]) + _dotd(a_sum, ws_ref[...])
           + _dotd(aea, we_ref[...]) + deg * b_ref[...])
    h = jnp.maximum(pre, 0.0) * _rowmask(i)
    h_ref[...] = h
    blk = jnp.concatenate([jnp.sum(h, axis=0, keepdims=True),
                           jnp.sum(h * h, axis=0, keepdims=True)], axis=0)

    @pl.when(i == 0)
    def _():
        stats_ref[...] = jnp.zeros_like(stats_ref)

    stats_ref[...] += blk


def _bn_apply(h, stats, g, be, i):
    mu = stats[0:1] * (1.0 / N)
    var = stats[1:2] * (1.0 / N) - mu * mu
    hn = (h - mu) / jnp.sqrt(var + EPS) * g + be
    return jnp.maximum(hn, 0.0) * _rowmask(i)


def _tc_bn1_body(h_ref, stats_ref, g_ref, be_ref, out_ref):
    out_ref[...] = _bn_apply(h_ref[...], stats_ref[...], g_ref[...],
                             be_ref[...], pl.program_id(0))


def _onehot(batch_blk):
    gidx = lax.broadcasted_iota(jnp.int32, (BR, G), 1)
    return (batch_blk == gidx).astype(jnp.float32)


def _tc_bn2_body(h_ref, stats_ref, g_ref, be_ref, batch_ref,
                 out_ref, pool_ref):
    i = pl.program_id(0)
    h2 = _bn_apply(h_ref[...], stats_ref[...], g_ref[...], be_ref[...], i)
    out_ref[...] = h2
    pool_blk = lax.dot_general(_onehot(batch_ref[...]), h2,
                               (((0,), (0,)), ((), ())),
                               preferred_element_type=jnp.float32)

    @pl.when(i == 0)
    def _():
        pool_ref[...] = jnp.zeros_like(pool_ref)

    pool_ref[...] += pool_blk


def _tc_mlp_body(h2_ref, pool_ref, batch_ref, wf1h_ref, wf1r_ref, bf1_ref,
                 wf2_ref, bf2_ref, out_ref):
    rep = _dotd(_onehot(batch_ref[...]), pool_ref[...])
    no = jnp.maximum(_dotd(h2_ref[...], wf1h_ref[...]) +
                     _dotd(rep, wf1r_ref[...]) + bf1_ref[...], 0.0)
    out_ref[...] = _dotd(no, wf2_ref[...]) + bf2_ref[...]


def _blk(shape, imap):
    return pl.BlockSpec(shape, imap)

_row = lambda i: (i, 0)
_fix = lambda i: (0, 0)
_row3 = lambda i: (0, i, 0)


def _build_tc(interpret=False):
    tc_mp = pl.pallas_call(
        _tc_mp_body,
        grid=(NBLK,),
        in_specs=[
            _blk((BR, D), _row),
            _blk((NC, BR, D), _row3),
            _blk((NC, BR, D), _row3),
            _blk((D, H), _fix),
            _blk((D, H), _fix),
            _blk((DE, H), _fix),
            _blk((1, H), _fix),
        ],
        out_specs=(_blk((BR, H), _row), _blk((2, H), _fix)),
        out_shape=(jax.ShapeDtypeStruct((NP, H), jnp.float32),
                   jax.ShapeDtypeStruct((2, H), jnp.float32)),
        interpret=interpret)

    tc_bn1 = pl.pallas_call(
        _tc_bn1_body,
        grid=(NBLK,),
        in_specs=[
            _blk((BR, H), _row),
            _blk((2, H), _fix),
            _blk((1, H), _fix),
            _blk((1, H), _fix),
        ],
        out_specs=_blk((BR, H), _row),
        out_shape=jax.ShapeDtypeStruct((NP, H), jnp.float32),
        interpret=interpret)

    tc_bn2 = pl.pallas_call(
        _tc_bn2_body,
        grid=(NBLK,),
        in_specs=[
            _blk((BR, H), _row),
            _blk((2, H), _fix),
            _blk((1, H), _fix),
            _blk((1, H), _fix),
            _blk((BR, 1), _row),
        ],
        out_specs=(_blk((BR, H), _row), _blk((G, H), _fix)),
        out_shape=(jax.ShapeDtypeStruct((NP, H), jnp.float32),
                   jax.ShapeDtypeStruct((G, H), jnp.float32)),
        interpret=interpret)

    tc_mlp = pl.pallas_call(
        _tc_mlp_body,
        grid=(NBLK,),
        in_specs=[
            _blk((BR, H), _row),
            _blk((G, H), _fix),
            _blk((BR, 1), _row),
            _blk((H, MLP_DIM), _fix),
            _blk((H, MLP_DIM), _fix),
            _blk((1, MLP_DIM), _fix),
            _blk((MLP_DIM, C), _fix),
            _blk((1, C), _fix),
        ],
        out_specs=_blk((BR, C), _row),
        out_shape=jax.ShapeDtypeStruct((NP, C), jnp.float32),
        interpret=interpret)
    return tc_mp, tc_bn1, tc_bn2, tc_mlp


_tc_mp, _tc_bn1, _tc_bn2, _tc_mlp = _build_tc()


def kernel(x, edge_index, edge_attr, batch, mask,
           W1, b1, g1, be1, W2, b2, g2, be2, Wf1, bf1, Wf2, bf2):
    del mask
    src = edge_index[0]
    dst = edge_index[1]
    pad_e = EP - E
    pad_ids = jnp.arange(pad_e, dtype=jnp.int32)
    src3 = jnp.concatenate([src, pad_ids % N]).reshape(NW, KB, BS)
    dst3 = jnp.concatenate(
        [dst, N + pad_ids % (NP - N)]).reshape(NW, KB, BS)
    eaug = jnp.pad(jnp.concatenate(
        [edge_attr, jnp.ones((E, 1), jnp.float32)], axis=1),
        ((0, pad_e), (0, D - DE - 1)))

    x_pad = jnp.concatenate([x, jnp.zeros((NP - N, D), jnp.float32)])
    batch2 = jnp.concatenate(
        [batch, jnp.full((NP - N,), G, jnp.int32)]).reshape(NP, 1)

    zrow = jnp.zeros((BS, D), jnp.float32)

    wd1 = W1[:, :D].T
    ws1 = W1[:, D:2 * D].T
    we1 = W1[:, 2 * D:].T
    wd2 = W2[:, :H].T
    ws2 = W2[:, H:2 * H].T
    we2 = W2[:, 2 * H:].T
    wf1h = Wf1[:, :H].T
    wf1r = Wf1[:, H:].T
    wf2 = Wf2.T

    sc_gather, sc_aug = _sc_passes()
    u1 = sc_aug(eaug, dst3, zrow)
    a1 = sc_gather(x_pad, src3, dst3, zrow)
    hp1, stats1 = _tc_mp(x_pad, a1, u1, wd1, ws1, we1, b1.reshape(1, H))
    h1 = _tc_bn1(hp1, stats1, g1.reshape(1, H), be1.reshape(1, H))
    a2 = sc_gather(h1, src3, dst3, zrow)
    hp2, stats2 = _tc_mp(h1, a2, u1, wd2, ws2, we2, b2.reshape(1, H))
    h2, pool = _tc_bn2(hp2, stats2, g2.reshape(1, H), be2.reshape(1, H),
                       batch2)
    out = _tc_mlp(h2, pool, batch2, wf1h, wf1r, bf1.reshape(1, MLP_DIM),
                  wf2, bf2.reshape(1, C))
    return out[:N]
